# initial kernel scaffold (unmeasured)
import jax
import jax.numpy as jnp
from jax import lax
from jax.experimental import pallas as pl
from jax.experimental.pallas import tpu as pltpu

T = 1024
D = 1024
F = 2048
E = 8
TSH = 512
ESH = 4
NEG = -1e30


def _mm(a, b):
    return lax.dot_general(
        a, b, (((1,), (0,)), ((), ())), preferred_element_type=jnp.float32
    )


def kernel(x, router, W1, W2):
    def body(x_ref, r_ref, w1_hbm, w2_hbm, out_ref,
             xfull, rfull, w1buf, w2buf, h_ref, part_ref, precv, brecv,
             send_sems, recv_sems, load_sems):
        p = lax.axis_index("x")
        q = lax.axis_index("y")
        xn = (1 - p, q)
        yn = (p, 1 - q)

        le0 = 2 * q
        ld_w1 = pltpu.make_async_copy(w1_hbm.at[le0], w1buf, load_sems.at[0])
        ld_w2 = pltpu.make_async_copy(w2_hbm.at[le0], w2buf, load_sems.at[1])
        ld_w1.start()
        ld_w2.start()

        cp_xl = pltpu.make_async_copy(x_ref, xfull.at[p], load_sems.at[2])
        cp_rl = pltpu.make_async_copy(r_ref, rfull.at[p], load_sems.at[3])
        cp_xl.start()
        cp_rl.start()

        bar = pltpu.get_barrier_semaphore()
        for nbr in (xn, yn):
            pl.semaphore_signal(bar, inc=1, device_id=nbr,
                                device_id_type=pl.DeviceIdType.MESH)
        pl.semaphore_wait(bar, 2)

        cp_x = pltpu.make_async_remote_copy(
            src_ref=x_ref, dst_ref=xfull.at[p],
            send_sem=send_sems.at[0], recv_sem=recv_sems.at[0],
            device_id=xn, device_id_type=pl.DeviceIdType.MESH)
        cp_r = pltpu.make_async_remote_copy(
            src_ref=r_ref, dst_ref=rfull.at[p],
            send_sem=send_sems.at[1], recv_sem=recv_sems.at[1],
            device_id=xn, device_id_type=pl.DeviceIdType.MESH)
        cp_x.start()
        cp_r.start()
        cp_x.wait()
        cp_r.wait()
        cp_xl.wait()
        cp_rl.wait()

        xf = xfull[...].reshape(T, D)
        g0 = _mm(xf, rfull[0])
        g1 = _mm(xf, rfull[1])

        idx4 = lax.broadcasted_iota(jnp.int32, (T, ESH), 1)
        m0 = jnp.max(g0, axis=1, keepdims=True)
        i0 = jnp.min(jnp.where(g0 >= m0, idx4, ESH), axis=1, keepdims=True)
        m1g = jnp.max(g1, axis=1, keepdims=True)
        i1g = jnp.min(jnp.where(g1 >= m1g, idx4, ESH), axis=1, keepdims=True)
        top1 = jnp.maximum(m0, m1g)
        i_top1 = jnp.where(m0 >= m1g, i0, i1g + 4)
        g0m = jnp.where((i_top1 < 4) & (idx4 == i_top1), NEG, g0)
        g1m = jnp.where((i_top1 >= 4) & (idx4 == i_top1 - 4), NEG, g1)
        m0b = jnp.max(g0m, axis=1, keepdims=True)
        i0b = jnp.min(jnp.where(g0m >= m0b, idx4, ESH), axis=1, keepdims=True)
        m1b = jnp.max(g1m, axis=1, keepdims=True)
        i1b = jnp.min(jnp.where(g1m >= m1b, idx4, ESH), axis=1, keepdims=True)
        top2 = jnp.maximum(m0b, m1b)
        i_top2 = jnp.where(m0b >= m1b, i0b, i1b + 4)
        e2 = jnp.exp(top2 - top1)
        w_top1 = 1.0 / (1.0 + e2)
        w_top2 = e2 / (1.0 + e2)

        def expert_weight(eg):
            return (jnp.where(i_top1 == eg, w_top1, 0.0)
                    + jnp.where(i_top2 == eg, w_top2, 0.0))

        xfb = xf.astype(jnp.bfloat16)

        ld_w1.wait()
        ld_w2.wait()
        h_ref[...] = jnp.maximum(
            _mm(xfb, w1buf[...].astype(jnp.bfloat16)), 0.0
        ).astype(jnp.bfloat16)
        part_ref[...] = (
            _mm(h_ref[...], w2buf[...].astype(jnp.bfloat16))
            * expert_weight(4 * p + 2 * q)
        ).reshape(2, TSH, D)

        le1 = 2 * q + 1
        ld_w1b = pltpu.make_async_copy(w1_hbm.at[le1], w1buf, load_sems.at[0])
        ld_w2b = pltpu.make_async_copy(w2_hbm.at[le1], w2buf, load_sems.at[1])
        ld_w1b.start()
        ld_w2b.start()
        ld_w1b.wait()
        ld_w2b.wait()
        h_ref[...] = jnp.maximum(
            _mm(xfb, w1buf[...].astype(jnp.bfloat16)), 0.0
        ).astype(jnp.bfloat16)
        part_ref[...] = part_ref[...] + (
            _mm(h_ref[...], w2buf[...].astype(jnp.bfloat16))
            * expert_weight(4 * p + 2 * q + 1)
        ).reshape(2, TSH, D)

        cp_a = pltpu.make_async_remote_copy(
            src_ref=part_ref.at[1 - p], dst_ref=precv,
            send_sem=send_sems.at[2], recv_sem=recv_sems.at[2],
            device_id=xn, device_id_type=pl.DeviceIdType.MESH)
        cp_a.start()
        cp_a.wait()
        out_ref[...] = part_ref[p] + precv[...]

        cp_b = pltpu.make_async_remote_copy(
            src_ref=out_ref, dst_ref=brecv,
            send_sem=send_sems.at[3], recv_sem=recv_sems.at[3],
            device_id=yn, device_id_type=pl.DeviceIdType.MESH)
        cp_b.start()
        cp_b.wait()
        out_ref[...] = out_ref[...] + brecv[...]

    return pl.pallas_call(
        body,
        out_shape=jax.ShapeDtypeStruct((TSH, D), jnp.float32),
        in_specs=[
            pl.BlockSpec(memory_space=pltpu.VMEM),
            pl.BlockSpec(memory_space=pltpu.VMEM),
            pl.BlockSpec(memory_space=pltpu.ANY),
            pl.BlockSpec(memory_space=pltpu.ANY),
        ],
        out_specs=pl.BlockSpec(memory_space=pltpu.VMEM),
        scratch_shapes=[
            pltpu.VMEM((2, TSH, D), jnp.float32),
            pltpu.VMEM((2, D, ESH), jnp.float32),
            pltpu.VMEM((D, F), jnp.float32),
            pltpu.VMEM((F, D), jnp.float32),
            pltpu.VMEM((T, F), jnp.bfloat16),
            pltpu.VMEM((2, TSH, D), jnp.float32),
            pltpu.VMEM((TSH, D), jnp.float32),
            pltpu.VMEM((TSH, D), jnp.float32),
            pltpu.SemaphoreType.DMA((4,)),
            pltpu.SemaphoreType.DMA((4,)),
            pltpu.SemaphoreType.DMA((4,)),
        ],
        compiler_params=pltpu.CompilerParams(collective_id=0),
    )(x, router, W1, W2)


# baseline (device time: 123165 ns/iter reference)
import jax
import jax.numpy as jnp
from jax import lax
from jax.experimental import pallas as pl
from jax.experimental.pallas import tpu as pltpu

T = 1024
D = 1024
F = 2048
E = 8
TSH = 512
ESH = 4
NEG = -1e30


def _mm(a, b, precision=None):
    return lax.dot_general(
        a, b, (((1,), (0,)), ((), ())),
        preferred_element_type=jnp.float32, precision=precision,
    )


def kernel(x, router, W1, W2):
    def body(x_ref, r_ref, w1_hbm, w2_hbm, out_ref,
             xfull, rfull, w1buf, w2buf, h_ref, part_ref, precv, brecv,
             send_sems, recv_sems, load_sems):
        p = lax.axis_index("x")
        q = lax.axis_index("y")
        xn = (1 - p, q)
        yn = (p, 1 - q)

        le0 = 2 * q
        ld_w1 = pltpu.make_async_copy(w1_hbm.at[le0], w1buf, load_sems.at[0])
        ld_w2 = pltpu.make_async_copy(w2_hbm.at[le0], w2buf, load_sems.at[1])
        ld_w1.start()
        ld_w2.start()

        cp_xl = pltpu.make_async_copy(x_ref, xfull.at[p], load_sems.at[2])
        cp_rl = pltpu.make_async_copy(r_ref, rfull.at[p], load_sems.at[3])
        cp_xl.start()
        cp_rl.start()

        bar = pltpu.get_barrier_semaphore()
        for nbr in (xn, yn):
            pl.semaphore_signal(bar, inc=1, device_id=nbr,
                                device_id_type=pl.DeviceIdType.MESH)
        pl.semaphore_wait(bar, 2)

        cp_x = pltpu.make_async_remote_copy(
            src_ref=x_ref, dst_ref=xfull.at[p],
            send_sem=send_sems.at[0], recv_sem=recv_sems.at[0],
            device_id=xn, device_id_type=pl.DeviceIdType.MESH)
        cp_r = pltpu.make_async_remote_copy(
            src_ref=r_ref, dst_ref=rfull.at[p],
            send_sem=send_sems.at[1], recv_sem=recv_sems.at[1],
            device_id=xn, device_id_type=pl.DeviceIdType.MESH)
        cp_x.start()
        cp_r.start()
        cp_x.wait()
        cp_r.wait()
        cp_xl.wait()
        cp_rl.wait()

        xf = xfull[...].reshape(T, D)
        g0 = _mm(xf, rfull[0], lax.Precision.HIGHEST)
        g1 = _mm(xf, rfull[1], lax.Precision.HIGHEST)

        idx4 = lax.broadcasted_iota(jnp.int32, (T, ESH), 1)
        m0 = jnp.max(g0, axis=1, keepdims=True)
        i0 = jnp.min(jnp.where(g0 >= m0, idx4, ESH), axis=1, keepdims=True)
        m1g = jnp.max(g1, axis=1, keepdims=True)
        i1g = jnp.min(jnp.where(g1 >= m1g, idx4, ESH), axis=1, keepdims=True)
        top1 = jnp.maximum(m0, m1g)
        i_top1 = jnp.where(m0 >= m1g, i0, i1g + 4)
        g0m = jnp.where((i_top1 < 4) & (idx4 == i_top1), NEG, g0)
        g1m = jnp.where((i_top1 >= 4) & (idx4 == i_top1 - 4), NEG, g1)
        m0b = jnp.max(g0m, axis=1, keepdims=True)
        i0b = jnp.min(jnp.where(g0m >= m0b, idx4, ESH), axis=1, keepdims=True)
        m1b = jnp.max(g1m, axis=1, keepdims=True)
        i1b = jnp.min(jnp.where(g1m >= m1b, idx4, ESH), axis=1, keepdims=True)
        top2 = jnp.maximum(m0b, m1b)
        i_top2 = jnp.where(m0b >= m1b, i0b, i1b + 4)
        e2 = jnp.exp(top2 - top1)
        w_top1 = 1.0 / (1.0 + e2)
        w_top2 = e2 / (1.0 + e2)

        def expert_weight(eg):
            return (jnp.where(i_top1 == eg, w_top1, 0.0)
                    + jnp.where(i_top2 == eg, w_top2, 0.0))

        xfb = xf.astype(jnp.bfloat16)

        ld_w1.wait()
        ld_w2.wait()
        h_ref[...] = jnp.maximum(
            _mm(xfb, w1buf[...].astype(jnp.bfloat16)), 0.0
        ).astype(jnp.bfloat16)
        part_ref[...] = (
            _mm(h_ref[...], w2buf[...].astype(jnp.bfloat16))
            * expert_weight(4 * p + 2 * q)
        ).reshape(2, TSH, D)

        le1 = 2 * q + 1
        ld_w1b = pltpu.make_async_copy(w1_hbm.at[le1], w1buf, load_sems.at[0])
        ld_w2b = pltpu.make_async_copy(w2_hbm.at[le1], w2buf, load_sems.at[1])
        ld_w1b.start()
        ld_w2b.start()
        ld_w1b.wait()
        ld_w2b.wait()
        h_ref[...] = jnp.maximum(
            _mm(xfb, w1buf[...].astype(jnp.bfloat16)), 0.0
        ).astype(jnp.bfloat16)
        part_ref[...] = part_ref[...] + (
            _mm(h_ref[...], w2buf[...].astype(jnp.bfloat16))
            * expert_weight(4 * p + 2 * q + 1)
        ).reshape(2, TSH, D)

        cp_a = pltpu.make_async_remote_copy(
            src_ref=part_ref.at[1 - p], dst_ref=precv,
            send_sem=send_sems.at[2], recv_sem=recv_sems.at[2],
            device_id=xn, device_id_type=pl.DeviceIdType.MESH)
        cp_a.start()
        cp_a.wait()
        out_ref[...] = part_ref[p] + precv[...]

        cp_b = pltpu.make_async_remote_copy(
            src_ref=out_ref, dst_ref=brecv,
            send_sem=send_sems.at[3], recv_sem=recv_sems.at[3],
            device_id=yn, device_id_type=pl.DeviceIdType.MESH)
        cp_b.start()
        cp_b.wait()
        out_ref[...] = out_ref[...] + brecv[...]

    return pl.pallas_call(
        body,
        out_shape=jax.ShapeDtypeStruct((TSH, D), jnp.float32),
        in_specs=[
            pl.BlockSpec(memory_space=pltpu.VMEM),
            pl.BlockSpec(memory_space=pltpu.VMEM),
            pl.BlockSpec(memory_space=pl.ANY),
            pl.BlockSpec(memory_space=pl.ANY),
        ],
        out_specs=pl.BlockSpec(memory_space=pltpu.VMEM),
        scratch_shapes=[
            pltpu.VMEM((2, TSH, D), jnp.float32),
            pltpu.VMEM((2, D, ESH), jnp.float32),
            pltpu.VMEM((D, F), jnp.float32),
            pltpu.VMEM((F, D), jnp.float32),
            pltpu.VMEM((T, F), jnp.bfloat16),
            pltpu.VMEM((2, TSH, D), jnp.float32),
            pltpu.VMEM((TSH, D), jnp.float32),
            pltpu.VMEM((TSH, D), jnp.float32),
            pltpu.SemaphoreType.DMA((4,)),
            pltpu.SemaphoreType.DMA((4,)),
            pltpu.SemaphoreType.DMA((4,)),
        ],
        compiler_params=pltpu.CompilerParams(
            collective_id=0, vmem_limit_bytes=56 * 1024 * 1024
        ),
    )(x, router, W1, W2)


# device time: 58714 ns/iter; 2.0977x vs baseline; 2.0977x over previous
import jax
import jax.numpy as jnp
from jax import lax
from jax.experimental import pallas as pl
from jax.experimental.pallas import tpu as pltpu

T = 1024
D = 1024
F = 2048
E = 8
TSH = 512
ESH = 4
NC = 2
CH = TSH // NC
NEG = -1e30


def _mm(a, b, precision=None):
    return lax.dot_general(
        a, b, (((1,), (0,)), ((), ())),
        preferred_element_type=jnp.float32, precision=precision,
    )


def kernel(x, router, W1, W2):
    def body(x_ref, r_ref, w1_hbm, w2_hbm, out_ref,
             xsend, xrecv, wsend, wrecv, rfull, w1buf, w2buf,
             part_other, part_mine, precv, bsend, brecv,
             send_sems, recv_sems, load_sems):
        p = lax.axis_index("x")
        q = lax.axis_index("y")
        xn = (1 - p, q)
        yn = (p, 1 - q)

        le0 = 2 * q
        le1 = 2 * q + 1
        lds = [
            pltpu.make_async_copy(w1_hbm.at[le0], w1buf.at[0], load_sems.at[0]),
            pltpu.make_async_copy(w2_hbm.at[le0], w2buf.at[0], load_sems.at[1]),
            pltpu.make_async_copy(w1_hbm.at[le1], w1buf.at[1], load_sems.at[2]),
            pltpu.make_async_copy(w2_hbm.at[le1], w2buf.at[1], load_sems.at[3]),
        ]
        for ld in lds:
            ld.start()
        cp_rl = pltpu.make_async_copy(r_ref, rfull.at[p], load_sems.at[4])
        cp_rl.start()

        bar = pltpu.get_barrier_semaphore()
        for nbr in (xn, yn):
            pl.semaphore_signal(bar, inc=1, device_id=nbr,
                                device_id_type=pl.DeviceIdType.MESH)
        pl.semaphore_wait(bar, 2)

        cp_r = pltpu.make_async_remote_copy(
            src_ref=r_ref, dst_ref=rfull.at[p],
            send_sem=send_sems.at[0], recv_sem=recv_sems.at[0],
            device_id=xn, device_id_type=pl.DeviceIdType.MESH)
        cp_r.start()
        xsend[...] = x_ref[...].astype(jnp.bfloat16)
        cp_x = pltpu.make_async_remote_copy(
            src_ref=xsend, dst_ref=xrecv,
            send_sem=send_sems.at[1], recv_sem=recv_sems.at[1],
            device_id=xn, device_id_type=pl.DeviceIdType.MESH)
        cp_x.start()

        cp_r.wait()
        cp_rl.wait()
        idx4 = lax.broadcasted_iota(jnp.int32, (TSH, ESH), 1)
        g0 = _mm(x_ref[...], rfull[0], lax.Precision.HIGHEST)
        g1 = _mm(x_ref[...], rfull[1], lax.Precision.HIGHEST)
        m0 = jnp.max(g0, axis=1, keepdims=True)
        i0 = jnp.min(jnp.where(g0 >= m0, idx4, ESH), axis=1, keepdims=True)
        m1 = jnp.max(g1, axis=1, keepdims=True)
        i1 = jnp.min(jnp.where(g1 >= m1, idx4, ESH), axis=1, keepdims=True)
        t1 = jnp.maximum(m0, m1)
        it1 = jnp.where(m0 >= m1, i0, i1 + 4)
        g0m = jnp.where((it1 < 4) & (idx4 == it1), NEG, g0)
        g1m = jnp.where((it1 >= 4) & (idx4 == it1 - 4), NEG, g1)
        m0b = jnp.max(g0m, axis=1, keepdims=True)
        i0b = jnp.min(jnp.where(g0m >= m0b, idx4, ESH), axis=1, keepdims=True)
        m1b = jnp.max(g1m, axis=1, keepdims=True)
        i1b = jnp.min(jnp.where(g1m >= m1b, idx4, ESH), axis=1, keepdims=True)
        t2 = jnp.maximum(m0b, m1b)
        it2 = jnp.where(m0b >= m1b, i0b, i1b + 4)
        e2 = jnp.exp(t2 - t1)
        wa = 1.0 / (1.0 + e2)
        wb = e2 / (1.0 + e2)

        def ew(eg):
            return (jnp.where(it1 == eg, wa, 0.0)
                    + jnp.where(it2 == eg, wb, 0.0))

        wsend[:, 0:1] = ew(4 * (1 - p) + le0).astype(jnp.bfloat16)
        wsend[:, 1:2] = ew(4 * (1 - p) + le1).astype(jnp.bfloat16)
        cp_w = pltpu.make_async_remote_copy(
            src_ref=wsend, dst_ref=wrecv,
            send_sem=send_sems.at[2], recv_sem=recv_sems.at[2],
            device_id=xn, device_id_type=pl.DeviceIdType.MESH)
        cp_w.start()

        def ffn(xb, slot):
            h = jnp.maximum(
                _mm(xb, w1buf[slot].astype(jnp.bfloat16)), 0.0
            ).astype(jnp.bfloat16)
            return _mm(h, w2buf[slot].astype(jnp.bfloat16))

        xmb = x_ref[...].astype(jnp.bfloat16)
        lds[0].wait()
        lds[1].wait()
        part_mine[...] = ffn(xmb, 0) * ew(4 * p + le0)

        cp_x.wait()
        cp_w.wait()
        lds[2].wait()
        lds[3].wait()
        cps_a = []
        for k in range(NC):
            ck = pl.ds(k * CH, CH)
            xok = xrecv[ck, :]
            ck_part = (ffn(xok, 0) * wrecv[ck, 0:1].astype(jnp.float32)
                       + ffn(xok, 1) * wrecv[ck, 1:2].astype(jnp.float32))
            part_other[ck, :] = ck_part.astype(jnp.bfloat16)
            cp_a = pltpu.make_async_remote_copy(
                src_ref=part_other.at[ck], dst_ref=precv.at[ck],
                send_sem=send_sems.at[3 + k], recv_sem=recv_sems.at[3 + k],
                device_id=xn, device_id_type=pl.DeviceIdType.MESH)
            cp_a.start()
            cps_a.append(cp_a)

        part_mine[...] = part_mine[...] + ffn(xmb, 1) * ew(4 * p + le1)

        cps_b = []
        for k in range(NC):
            ck = pl.ds(k * CH, CH)
            cps_a[k].wait()
            s = part_mine[ck, :] + precv[ck, :].astype(jnp.float32)
            out_ref[ck, :] = s
            bsend[ck, :] = s.astype(jnp.bfloat16)
            cp_b = pltpu.make_async_remote_copy(
                src_ref=bsend.at[ck], dst_ref=brecv.at[ck],
                send_sem=send_sems.at[3 + NC + k],
                recv_sem=recv_sems.at[3 + NC + k],
                device_id=yn, device_id_type=pl.DeviceIdType.MESH)
            cp_b.start()
            cps_b.append(cp_b)
        for k in range(NC):
            ck = pl.ds(k * CH, CH)
            cps_b[k].wait()
            out_ref[ck, :] = out_ref[ck, :] + brecv[ck, :].astype(jnp.float32)

    return pl.pallas_call(
        body,
        out_shape=jax.ShapeDtypeStruct((TSH, D), jnp.float32),
        in_specs=[
            pl.BlockSpec(memory_space=pltpu.VMEM),
            pl.BlockSpec(memory_space=pltpu.VMEM),
            pl.BlockSpec(memory_space=pl.ANY),
            pl.BlockSpec(memory_space=pl.ANY),
        ],
        out_specs=pl.BlockSpec(memory_space=pltpu.VMEM),
        scratch_shapes=[
            pltpu.VMEM((TSH, D), jnp.bfloat16),
            pltpu.VMEM((TSH, D), jnp.bfloat16),
            pltpu.VMEM((TSH, 128), jnp.bfloat16),
            pltpu.VMEM((TSH, 128), jnp.bfloat16),
            pltpu.VMEM((2, D, ESH), jnp.float32),
            pltpu.VMEM((2, D, F), jnp.float32),
            pltpu.VMEM((2, F, D), jnp.float32),
            pltpu.VMEM((TSH, D), jnp.bfloat16),
            pltpu.VMEM((TSH, D), jnp.float32),
            pltpu.VMEM((TSH, D), jnp.bfloat16),
            pltpu.VMEM((TSH, D), jnp.bfloat16),
            pltpu.VMEM((TSH, D), jnp.bfloat16),
            pltpu.SemaphoreType.DMA((3 + 2 * NC,)),
            pltpu.SemaphoreType.DMA((3 + 2 * NC,)),
            pltpu.SemaphoreType.DMA((5,)),
        ],
        compiler_params=pltpu.CompilerParams(
            collective_id=0, vmem_limit_bytes=60 * 1024 * 1024
        ),
    )(x, router, W1, W2)
